# baseline (speedup 1.0000x reference)
import functools

import jax
import jax.numpy as jnp
from jax import lax
from jax.experimental import pallas as pl
from jax.experimental.pallas import tpu as pltpu
from jax.experimental.pallas import tpu_sc as plsc

TOPK = 256


def _powsum_body(x_ref, o_ref):
    x = x_ref[...]
    B, RB, C = x.shape
    sq = x * x
    acc = sq[..., 0:128]
    for c in range(1, C // 128):
        acc = acc + sq[..., 128 * c:128 * (c + 1)]
    acc = acc.reshape(B * RB, 128)
    at = jnp.transpose(acc)
    t = at[0:8]
    for j in range(1, 16):
        t = t + at[8 * j:8 * j + 8]
    t = t[0:4] + t[4:8]
    t = t[0:2] + t[2:4]
    t = t[0:1] + t[1:2]
    o_ref[...] = t[None]


def _powsum(x):
    B, N, C = x.shape
    RB = 512
    nw = N // RB
    psw = pl.pallas_call(
        _powsum_body,
        grid=(nw,),
        in_specs=[pl.BlockSpec((B, RB, C), lambda r: (0, r, 0))],
        out_specs=pl.BlockSpec((1, 1, B * RB), lambda r: (r, 0, 0)),
        out_shape=jax.ShapeDtypeStruct((nw, 1, B * RB), jnp.float32),
    )(x)
    return psw.reshape(nw, B, RB).transpose(1, 0, 2).reshape(B, N)


def _topk_gather_sc(ps, xf):
    B, N = ps.shape
    C = xf.shape[1]
    K = TOPK
    NV = N // 16
    mesh = plsc.VectorSubcoreMesh(core_axis_name="c", subcore_axis_name="s")
    i32 = jnp.int32

    @functools.partial(
        pl.kernel,
        mesh=mesh,
        out_type=jax.ShapeDtypeStruct((B * K, C), jnp.float32),
        scratch_types=[
            pltpu.VMEM((N,), jnp.float32),
            pltpu.VMEM((256,), i32),
            pltpu.VMEM((K,), i32),
            pltpu.VMEM((K,), i32),
            pltpu.VMEM((16,), i32),
            pltpu.VMEM((16,), i32),
            pltpu.VMEM((K // 8, C), jnp.float32),
            pltpu.VMEM((K // 8,), i32),
            pltpu.VMEM_SHARED((2 * K,), i32),
            pltpu.VMEM_SHARED((2 * K,), i32),
            pltpu.VMEM_SHARED((2 * K,), i32),
            pltpu.SemaphoreType.DMA,
        ],
        compiler_params=pltpu.CompilerParams(needs_layout_passes=False),
    )
    def k(ps_hbm, x_hbm, out_hbm, vals_v, hist_v, cand_b, cand_i, pos_v,
          gv_v, rows_v, gidx_v, sh_bits, sh_idx, sh_out, sem):
        c = lax.axis_index("c")
        s = lax.axis_index("s")
        g = s // 8
        q = s % 8
        b = c * 2 + g
        ones = jnp.ones((16,), i32)
        zeros = jnp.zeros((16,), i32)
        lanes = lax.iota(i32, 16)

        @pl.when(q == 0)
        def _lead():
            pltpu.sync_copy(ps_hbm.at[pl.ds(b * N, N)], vals_v)
            for h in range(K // 16):
                cand_b[pl.ds(h * 16, 16)] = zeros
            pltpu.sync_copy(cand_b, sh_out.at[pl.ds(g * K, K)])

            thr = jnp.asarray(0, i32)
            n_gt = jnp.asarray(0, i32)
            for r in range(4):
                shift = 24 - 8 * r
                for h in range(16):
                    hist_v[pl.ds(h * 16, 16)] = zeros
                phi = (thr >> (shift + 8)) if r else None

                def hbody(i, carry, phi=phi, shift=shift, r=r):
                    v = plsc.bitcast(vals_v[pl.ds(i * 16, 16)], i32)
                    byte = (v >> shift) & 255
                    if r:
                        m = (v >> (shift + 8)) == jnp.full((16,), phi, i32)
                        plsc.addupdate_scatter(hist_v, [byte], ones, mask=m)
                    else:
                        plsc.addupdate_scatter(hist_v, [byte], ones)
                    return carry
                lax.fori_loop(0, NV, hbody, jnp.asarray(0, i32))

                target = jnp.asarray(K, i32) - n_gt
                carry = jnp.asarray(0, i32)
                found = jnp.asarray(0, i32)
                hbyte = jnp.asarray(0, i32)
                ngt_new = jnp.asarray(0, i32)
                for j in range(15, -1, -1):
                    w = hist_v[pl.ds(j * 16, 16)]
                    rw = lax.rev(w, (0,))
                    cs = plsc.cumsum(rw)
                    tot = cs[15]
                    gcum = jnp.full((16,), carry, i32) + cs
                    hit = gcum >= jnp.full((16,), target, i32)
                    lstar = jnp.min(
                        jnp.where(hit, lanes, jnp.full((16,), 16, i32)))
                    vreg_hit = jnp.logical_and(found == 0,
                                               carry + tot >= target)
                    msel = lanes == jnp.full((16,), lstar, i32)
                    cs_at = jnp.sum(jnp.where(msel, cs, zeros))
                    w_at = jnp.sum(jnp.where(msel, rw, zeros))
                    hbyte = jnp.where(vreg_hit, 16 * j + 15 - lstar, hbyte)
                    ngt_new = jnp.where(vreg_hit, n_gt + carry + cs_at - w_at,
                                        ngt_new)
                    found = jnp.where(vreg_hit, 1, found)
                    carry = carry + tot
                thr = thr | (hbyte << shift)
                n_gt = ngt_new

            quota = jnp.asarray(K, i32) - n_gt
            thr_v = jnp.full((16,), thr, i32)

            def cbody(i, carry):
                off_gt, eq_tk = carry
                v = plsc.bitcast(vals_v[pl.ds(i * 16, 16)], i32)
                ivec = lanes + i * 16
                kmax = jnp.full((16,), K - 1, i32)
                m_gt = v > thr_v
                c_gt = plsc.cumsum(jnp.where(m_gt, ones, zeros))
                pos = jnp.clip(jnp.full((16,), off_gt, i32) + c_gt - ones,
                               zeros, kmax)
                plsc.store_scatter(cand_b, [pos], v, mask=m_gt)
                plsc.store_scatter(cand_i, [pos], ivec, mask=m_gt)
                m_eq = v == thr_v
                c_eq = plsc.cumsum(jnp.where(m_eq, ones, zeros))
                m_tk = jnp.logical_and(
                    m_eq, (jnp.full((16,), eq_tk, i32) + c_eq)
                    <= jnp.full((16,), quota, i32))
                pos2 = jnp.clip(
                    jnp.full((16,), n_gt + eq_tk, i32) + c_eq - ones,
                    zeros, kmax)
                plsc.store_scatter(cand_b, [pos2], v, mask=m_tk)
                plsc.store_scatter(cand_i, [pos2], ivec, mask=m_tk)
                return (off_gt + c_gt[15],
                        jnp.minimum(eq_tk + c_eq[15], quota))
            lax.fori_loop(0, NV, cbody, (jnp.asarray(0, i32),
                                         jnp.asarray(0, i32)))
            pltpu.sync_copy(cand_b, sh_bits.at[pl.ds(g * K, K)])
            pltpu.sync_copy(cand_i, sh_idx.at[pl.ds(g * K, K)])

        plsc.subcore_barrier()

        pltpu.sync_copy(sh_bits.at[pl.ds(g * K, K)], cand_b)
        pltpu.sync_copy(sh_idx.at[pl.ds(g * K, K)], cand_i)
        for av in range(2):
            abits = cand_b[pl.ds(q * 32 + av * 16, 16)]
            aidx = cand_i[pl.ds(q * 32 + av * 16, 16)]

            def rbody(j, rank, abits=abits, aidx=aidx):
                bb = cand_b[pl.ds(j * 16, 16)]
                bi = cand_i[pl.ds(j * 16, 16)]
                for lb in range(16):
                    bbv = jnp.full((16,), bb[lb], i32)
                    biv = jnp.full((16,), bi[lb], i32)
                    m = jnp.logical_or(
                        bbv > abits,
                        jnp.logical_and(bbv == abits, biv < aidx))
                    rank = rank + jnp.where(m, ones, zeros)
                return rank
            rank = lax.fori_loop(0, K // 16, rbody, zeros)
            pos_v[...] = rank + g * K
            gv_v[...] = aidx + b * N
            pltpu.sync_copy(gv_v, sh_out.at[pos_v], add=True)

        plsc.subcore_barrier()

        rpw = K // 8
        pltpu.sync_copy(sh_out.at[pl.ds(g * K + q * rpw, rpw)], gidx_v)
        pltpu.async_copy(x_hbm.at[gidx_v], rows_v, sem).wait()
        pltpu.sync_copy(rows_v, out_hbm.at[pl.ds(b * K + q * rpw, rpw)])

    return k(ps.reshape(B * N), xf)


def kernel(x):
    B, N, C = x.shape
    ps = _powsum(x)
    out = _topk_gather_sc(ps, x.reshape(B * N, C))
    return out.reshape(B, TOPK, C)


# speedup vs baseline: 1.0320x; 1.0320x over previous
import functools

import jax
import jax.numpy as jnp
from jax import lax
from jax.experimental import pallas as pl
from jax.experimental.pallas import tpu as pltpu
from jax.experimental.pallas import tpu_sc as plsc

TOPK = 256


def _powsum_body(x_ref, o_ref):
    x = x_ref[...]
    B, RB, C = x.shape
    sq = x * x
    acc = sq[..., 0:128]
    for c in range(1, C // 128):
        acc = acc + sq[..., 128 * c:128 * (c + 1)]
    acc = acc.reshape(B * RB, 128)
    at = jnp.transpose(acc)
    t = at[0:8]
    for j in range(1, 16):
        t = t + at[8 * j:8 * j + 8]
    t = t[0:4] + t[4:8]
    t = t[0:2] + t[2:4]
    t = t[0:1] + t[1:2]
    o_ref[...] = t[None]


def _powsum_windows(x):
    B, N, C = x.shape
    RB = 512
    nw = N // RB
    psw = pl.pallas_call(
        _powsum_body,
        grid=(nw,),
        in_specs=[pl.BlockSpec((B, RB, C), lambda r: (0, r, 0))],
        out_specs=pl.BlockSpec((1, 1, B * RB), lambda r: (r, 0, 0)),
        out_shape=jax.ShapeDtypeStruct((nw, 1, B * RB), jnp.float32),
    )(x)
    return psw.reshape(nw * B * RB)


def _topk_gather_sc(psw, xf, B, N):
    C = xf.shape[1]
    K = TOPK
    RB = 512
    NV = N // 16
    TV = NV // 8
    mesh = plsc.VectorSubcoreMesh(core_axis_name="c", subcore_axis_name="s")
    i32 = jnp.int32

    @functools.partial(
        pl.kernel,
        mesh=mesh,
        out_type=jax.ShapeDtypeStruct((B * K, C), jnp.float32),
        scratch_types=[
            pltpu.VMEM((N,), jnp.float32),
            pltpu.VMEM((256,), i32),
            pltpu.VMEM((2048,), i32),
            pltpu.VMEM((K,), i32),
            pltpu.VMEM((K,), i32),
            pltpu.VMEM((16,), i32),
            pltpu.VMEM((16,), i32),
            pltpu.VMEM((K // 8, C), jnp.float32),
            pltpu.VMEM((K // 8,), i32),
            pltpu.VMEM_SHARED((4 * 2 * 8 * 256,), i32),
            pltpu.VMEM_SHARED((2 * K,), i32),
            pltpu.VMEM_SHARED((2 * K,), i32),
            pltpu.VMEM_SHARED((2 * K,), i32),
            pltpu.SemaphoreType.DMA,
        ],
        compiler_params=pltpu.CompilerParams(needs_layout_passes=False),
    )
    def k(ps_hbm, x_hbm, out_hbm, vals_v, hist_v, hbuf, cand_b, cand_i,
          pos_v, gv_v, rows_v, gidx_v, sh_hist, sh_bits, sh_idx, sh_out,
          sem):
        c = lax.axis_index("c")
        s = lax.axis_index("s")
        g = s // 8
        q = s % 8
        b = c * 2 + g
        ones = jnp.ones((16,), i32)
        zeros = jnp.zeros((16,), i32)
        lanes = lax.iota(i32, 16)

        for r in range(N // RB):
            pltpu.sync_copy(ps_hbm.at[pl.ds(r * B * RB + b * RB, RB)],
                            vals_v.at[pl.ds(r * RB, RB)])

        @pl.when(q == 0)
        def _zero():
            for h in range(K // 16):
                cand_b[pl.ds(h * 16, 16)] = zeros
            pltpu.sync_copy(cand_b, sh_out.at[pl.ds(g * K, K)])

        thr = jnp.asarray(0, i32)
        n_gt = jnp.asarray(0, i32)
        for r in range(4):
            shift = 24 - 8 * r
            for h in range(16):
                hist_v[pl.ds(h * 16, 16)] = zeros
            phi = (thr >> (shift + 8)) if r else None
            for kk in range(TV):
                base = (q * TV + kk) * 16
                v = plsc.bitcast(vals_v[pl.ds(base, 16)], i32)
                byte = (v >> shift) & 255
                if r:
                    m = (v >> (shift + 8)) == jnp.full((16,), phi, i32)
                    plsc.addupdate_scatter(hist_v, [byte], ones, mask=m)
                else:
                    plsc.addupdate_scatter(hist_v, [byte], ones)
            pltpu.sync_copy(
                hist_v, sh_hist.at[pl.ds(((r * 2 + g) * 8 + q) * 256, 256)])
            plsc.subcore_barrier()
            pltpu.sync_copy(sh_hist.at[pl.ds((r * 2 + g) * 2048, 2048)],
                            hbuf)
            target = jnp.asarray(K, i32) - n_gt
            carry = jnp.asarray(0, i32)
            found = jnp.asarray(0, i32)
            hbyte = jnp.asarray(0, i32)
            ngt_new = jnp.asarray(0, i32)
            for j in range(15, -1, -1):
                w = hbuf[pl.ds(j * 16, 16)]
                for t in range(1, 8):
                    w = w + hbuf[pl.ds(t * 256 + j * 16, 16)]
                rw = lax.rev(w, (0,))
                cs = plsc.cumsum(rw)
                tot = cs[15]
                gcum = jnp.full((16,), carry, i32) + cs
                hit = gcum >= jnp.full((16,), target, i32)
                lstar = jnp.min(
                    jnp.where(hit, lanes, jnp.full((16,), 16, i32)))
                vreg_hit = jnp.logical_and(found == 0,
                                           carry + tot >= target)
                msel = lanes == jnp.full((16,), lstar, i32)
                cs_at = jnp.sum(jnp.where(msel, cs, zeros))
                w_at = jnp.sum(jnp.where(msel, rw, zeros))
                hbyte = jnp.where(vreg_hit, 16 * j + 15 - lstar, hbyte)
                ngt_new = jnp.where(vreg_hit, n_gt + carry + cs_at - w_at,
                                    ngt_new)
                found = jnp.where(vreg_hit, 1, found)
                carry = carry + tot
            thr = thr | (hbyte << shift)
            n_gt = ngt_new

        quota = jnp.asarray(K, i32) - n_gt
        thr_v = jnp.full((16,), thr, i32)

        @pl.when(q == 0)
        def _lead():
            def cbody(i, carry):
                off_gt, eq_tk = carry
                v = plsc.bitcast(vals_v[pl.ds(i * 16, 16)], i32)
                ivec = lanes + i * 16
                kmax = jnp.full((16,), K - 1, i32)
                m_gt = v > thr_v
                c_gt = plsc.cumsum(jnp.where(m_gt, ones, zeros))
                pos = jnp.clip(jnp.full((16,), off_gt, i32) + c_gt - ones,
                               zeros, kmax)
                plsc.store_scatter(cand_b, [pos], v, mask=m_gt)
                plsc.store_scatter(cand_i, [pos], ivec, mask=m_gt)
                m_eq = v == thr_v
                c_eq = plsc.cumsum(jnp.where(m_eq, ones, zeros))
                m_tk = jnp.logical_and(
                    m_eq, (jnp.full((16,), eq_tk, i32) + c_eq)
                    <= jnp.full((16,), quota, i32))
                pos2 = jnp.clip(
                    jnp.full((16,), n_gt + eq_tk, i32) + c_eq - ones,
                    zeros, kmax)
                plsc.store_scatter(cand_b, [pos2], v, mask=m_tk)
                plsc.store_scatter(cand_i, [pos2], ivec, mask=m_tk)
                return (off_gt + c_gt[15],
                        jnp.minimum(eq_tk + c_eq[15], quota))
            lax.fori_loop(0, NV, cbody, (jnp.asarray(0, i32),
                                         jnp.asarray(0, i32)))
            pltpu.sync_copy(cand_b, sh_bits.at[pl.ds(g * K, K)])
            pltpu.sync_copy(cand_i, sh_idx.at[pl.ds(g * K, K)])

        plsc.subcore_barrier()

        pltpu.sync_copy(sh_bits.at[pl.ds(g * K, K)], cand_b)
        pltpu.sync_copy(sh_idx.at[pl.ds(g * K, K)], cand_i)
        for av in range(2):
            abits = cand_b[pl.ds(q * 32 + av * 16, 16)]
            aidx = cand_i[pl.ds(q * 32 + av * 16, 16)]

            def rbody(j, rank, abits=abits, aidx=aidx):
                bb = cand_b[pl.ds(j * 16, 16)]
                bi = cand_i[pl.ds(j * 16, 16)]
                for lb in range(16):
                    bbv = jnp.full((16,), bb[lb], i32)
                    biv = jnp.full((16,), bi[lb], i32)
                    m = jnp.logical_or(
                        bbv > abits,
                        jnp.logical_and(bbv == abits, biv < aidx))
                    rank = rank + jnp.where(m, ones, zeros)
                return rank
            rank = lax.fori_loop(0, K // 16, rbody, zeros)
            pos_v[...] = rank + g * K
            gv_v[...] = aidx + b * N
            pltpu.sync_copy(gv_v, sh_out.at[pos_v], add=True)

        plsc.subcore_barrier()

        rpw = K // 8
        pltpu.sync_copy(sh_out.at[pl.ds(g * K + q * rpw, rpw)], gidx_v)
        pltpu.async_copy(x_hbm.at[gidx_v], rows_v, sem).wait()
        pltpu.sync_copy(rows_v, out_hbm.at[pl.ds(b * K + q * rpw, rpw)])

    return k(psw, xf)


def kernel(x):
    B, N, C = x.shape
    psw = _powsum_windows(x)
    out = _topk_gather_sc(psw, x.reshape(B * N, C), B, N)
    return out.reshape(B, TOPK, C)


# speedup vs baseline: 1.1286x; 1.0936x over previous
import functools

import jax
import jax.numpy as jnp
from jax import lax
from jax.experimental import pallas as pl
from jax.experimental.pallas import tpu as pltpu
from jax.experimental.pallas import tpu_sc as plsc

TOPK = 256


def _powsum_body(x_ref, o_ref):
    x = x_ref[...]
    B, RB, C = x.shape
    sq = x * x
    acc = sq[..., 0:128]
    for c in range(1, C // 128):
        acc = acc + sq[..., 128 * c:128 * (c + 1)]
    acc = acc.reshape(B * RB, 128)
    at = jnp.transpose(acc)
    t = at[0:8]
    for j in range(1, 16):
        t = t + at[8 * j:8 * j + 8]
    t = t[0:4] + t[4:8]
    t = t[0:2] + t[2:4]
    t = t[0:1] + t[1:2]
    o_ref[...] = t[None]


def _powsum_windows(x):
    B, N, C = x.shape
    RB = 512
    nw = N // RB
    psw = pl.pallas_call(
        _powsum_body,
        grid=(nw,),
        in_specs=[pl.BlockSpec((B, RB, C), lambda r: (0, r, 0))],
        out_specs=pl.BlockSpec((1, 1, B * RB), lambda r: (r, 0, 0)),
        out_shape=jax.ShapeDtypeStruct((nw, 1, B * RB), jnp.float32),
    )(x)
    return psw.reshape(nw * B * RB)


def _topk_gather_sc(psw, xf, B, N):
    C = xf.shape[1]
    K = TOPK
    RB = 512
    NV = N // 16
    TV = NV // 8
    mesh = plsc.VectorSubcoreMesh(core_axis_name="c", subcore_axis_name="s")
    i32 = jnp.int32

    @functools.partial(
        pl.kernel,
        mesh=mesh,
        out_type=jax.ShapeDtypeStruct((B * K, C), jnp.float32),
        scratch_types=[
            pltpu.VMEM((N,), jnp.float32),
            pltpu.VMEM((256,), i32),
            pltpu.VMEM((2048,), i32),
            pltpu.VMEM((K,), i32),
            pltpu.VMEM((K,), i32),
            pltpu.VMEM((16,), i32),
            pltpu.VMEM((16,), i32),
            pltpu.VMEM((K // 8, C), jnp.float32),
            pltpu.VMEM((K // 8,), i32),
            pltpu.VMEM_SHARED((4 * 2 * 8 * 256,), i32),
            pltpu.VMEM_SHARED((2 * K,), i32),
            pltpu.VMEM_SHARED((2 * K,), i32),
            pltpu.VMEM_SHARED((2 * K,), i32),
            pltpu.SemaphoreType.DMA,
        ],
        compiler_params=pltpu.CompilerParams(needs_layout_passes=False),
    )
    def k(ps_hbm, x_hbm, out_hbm, vals_v, hist_v, hbuf, cand_b, cand_i,
          pos_v, gv_v, rows_v, gidx_v, sh_hist, sh_bits, sh_idx, sh_out,
          sem):
        c = lax.axis_index("c")
        s = lax.axis_index("s")
        g = s // 8
        q = s % 8
        b = c * 2 + g
        ones = jnp.ones((16,), i32)
        zeros = jnp.zeros((16,), i32)
        lanes = lax.iota(i32, 16)

        @pl.when(q != 0)
        def _load_slice():
            pltpu.sync_copy(ps_hbm.at[pl.ds(q * B * RB + b * RB, RB)],
                            vals_v.at[pl.ds(0, RB)])

        @pl.when(q == 0)
        def _load_full():
            cps = [pltpu.async_copy(
                ps_hbm.at[pl.ds(r * B * RB + b * RB, RB)],
                vals_v.at[pl.ds(r * RB, RB)], sem)
                for r in range(N // RB)]
            for cp in cps:
                cp.wait()

        @pl.when(q == 0)
        def _zero():
            for h in range(K // 16):
                cand_b[pl.ds(h * 16, 16)] = zeros
            pltpu.sync_copy(cand_b, sh_out.at[pl.ds(g * K, K)])

        thr = jnp.asarray(0, i32)
        n_gt = jnp.asarray(0, i32)
        for r in range(4):
            shift = 24 - 8 * r
            for h in range(16):
                hist_v[pl.ds(h * 16, 16)] = zeros
            phi = (thr >> (shift + 8)) if r else None
            for kk in range(TV):
                base = kk * 16
                v = plsc.bitcast(vals_v[pl.ds(base, 16)], i32)
                byte = (v >> shift) & 255
                if r:
                    m = (v >> (shift + 8)) == jnp.full((16,), phi, i32)
                    plsc.addupdate_scatter(hist_v, [byte], ones, mask=m)
                else:
                    plsc.addupdate_scatter(hist_v, [byte], ones)
            pltpu.sync_copy(
                hist_v, sh_hist.at[pl.ds(((r * 2 + g) * 8 + q) * 256, 256)])
            plsc.subcore_barrier()
            pltpu.sync_copy(sh_hist.at[pl.ds((r * 2 + g) * 2048, 2048)],
                            hbuf)
            target = jnp.asarray(K, i32) - n_gt
            carry = jnp.asarray(0, i32)
            found = jnp.asarray(0, i32)
            hbyte = jnp.asarray(0, i32)
            ngt_new = jnp.asarray(0, i32)
            for j in range(15, -1, -1):
                w = hbuf[pl.ds(j * 16, 16)]
                for t in range(1, 8):
                    w = w + hbuf[pl.ds(t * 256 + j * 16, 16)]
                rw = lax.rev(w, (0,))
                cs = plsc.cumsum(rw)
                tot = cs[15]
                gcum = jnp.full((16,), carry, i32) + cs
                hit = gcum >= jnp.full((16,), target, i32)
                lstar = jnp.min(
                    jnp.where(hit, lanes, jnp.full((16,), 16, i32)))
                vreg_hit = jnp.logical_and(found == 0,
                                           carry + tot >= target)
                msel = lanes == jnp.full((16,), lstar, i32)
                cs_at = jnp.sum(jnp.where(msel, cs, zeros))
                w_at = jnp.sum(jnp.where(msel, rw, zeros))
                hbyte = jnp.where(vreg_hit, 16 * j + 15 - lstar, hbyte)
                ngt_new = jnp.where(vreg_hit, n_gt + carry + cs_at - w_at,
                                    ngt_new)
                found = jnp.where(vreg_hit, 1, found)
                carry = carry + tot
            thr = thr | (hbyte << shift)
            n_gt = ngt_new

        quota = jnp.asarray(K, i32) - n_gt
        thr_v = jnp.full((16,), thr, i32)

        @pl.when(q == 0)
        def _lead():
            def cbody(i, carry):
                off_gt, eq_tk = carry
                v = plsc.bitcast(vals_v[pl.ds(i * 16, 16)], i32)
                ivec = lanes + i * 16
                kmax = jnp.full((16,), K - 1, i32)
                m_gt = v > thr_v
                c_gt = plsc.cumsum(jnp.where(m_gt, ones, zeros))
                pos = jnp.clip(jnp.full((16,), off_gt, i32) + c_gt - ones,
                               zeros, kmax)
                plsc.store_scatter(cand_b, [pos], v, mask=m_gt)
                plsc.store_scatter(cand_i, [pos], ivec, mask=m_gt)
                m_eq = v == thr_v
                c_eq = plsc.cumsum(jnp.where(m_eq, ones, zeros))
                m_tk = jnp.logical_and(
                    m_eq, (jnp.full((16,), eq_tk, i32) + c_eq)
                    <= jnp.full((16,), quota, i32))
                pos2 = jnp.clip(
                    jnp.full((16,), n_gt + eq_tk, i32) + c_eq - ones,
                    zeros, kmax)
                plsc.store_scatter(cand_b, [pos2], v, mask=m_tk)
                plsc.store_scatter(cand_i, [pos2], ivec, mask=m_tk)
                return (off_gt + c_gt[15],
                        jnp.minimum(eq_tk + c_eq[15], quota))
            lax.fori_loop(0, NV, cbody, (jnp.asarray(0, i32),
                                         jnp.asarray(0, i32)))
            pltpu.sync_copy(cand_b, sh_bits.at[pl.ds(g * K, K)])
            pltpu.sync_copy(cand_i, sh_idx.at[pl.ds(g * K, K)])

        plsc.subcore_barrier()

        pltpu.sync_copy(sh_bits.at[pl.ds(g * K, K)], cand_b)
        pltpu.sync_copy(sh_idx.at[pl.ds(g * K, K)], cand_i)
        for av in range(2):
            abits = cand_b[pl.ds(q * 32 + av * 16, 16)]
            aidx = cand_i[pl.ds(q * 32 + av * 16, 16)]

            def rbody(j, rank, abits=abits, aidx=aidx):
                bb = cand_b[pl.ds(j * 16, 16)]
                bi = cand_i[pl.ds(j * 16, 16)]
                for lb in range(16):
                    bbv = jnp.full((16,), bb[lb], i32)
                    biv = jnp.full((16,), bi[lb], i32)
                    m = jnp.logical_or(
                        bbv > abits,
                        jnp.logical_and(bbv == abits, biv < aidx))
                    rank = rank + jnp.where(m, ones, zeros)
                return rank
            rank = lax.fori_loop(0, K // 16, rbody, zeros)
            pos_v[...] = rank + g * K
            gv_v[...] = aidx + b * N
            pltpu.sync_copy(gv_v, sh_out.at[pos_v], add=True)

        plsc.subcore_barrier()

        rpw = K // 8
        pltpu.sync_copy(sh_out.at[pl.ds(g * K + q * rpw, rpw)], gidx_v)
        pltpu.async_copy(x_hbm.at[gidx_v], rows_v, sem).wait()
        pltpu.sync_copy(rows_v, out_hbm.at[pl.ds(b * K + q * rpw, rpw)])

    return k(psw, xf)


def kernel(x):
    B, N, C = x.shape
    psw = _powsum_windows(x)
    out = _topk_gather_sc(psw, x.reshape(B * N, C), B, N)
    return out.reshape(B, TOPK, C)
